# baseline (speedup 1.0000x reference)
import functools
import math

import jax
import jax.numpy as jnp
from jax import lax
from jax.experimental import pallas as pl
from jax.experimental.pallas import tpu as pltpu
from jax.experimental.pallas import tpu_sc as plsc

N = 10000
E = 320000
D = 128
WIDTH = 136
CB = 40
NCHUNKS = E // CB
NW = 32
CPW = NCHUNKS // NW
NPAIRS = CPW // 2
NSUB = 16
ROWS_PER_SUB = N // NSUB
INV_SQRT_C = 1.0 / math.sqrt(float(D))
EPS = 1e-16

RB = 512
GRID = (N + RB - 1) // RB


def _proj_first_body(x_ref, w_ref, b_ref, q_ref, kv_ref, s_ref):
    o = jnp.dot(x_ref[...], w_ref[...], preferred_element_type=jnp.float32)
    o = o + b_ref[...]
    q_ref[...] = o[:, :D]
    kv_ref[...] = o[:, D:3 * D]
    s_ref[...] = o[:, 3 * D:]


def _proj_first(x, wcat, bcat):
    return pl.pallas_call(
        _proj_first_body,
        grid=(GRID,),
        in_specs=[
            pl.BlockSpec((RB, D), lambda i: (i, 0)),
            pl.BlockSpec((D, 4 * D), lambda i: (0, 0)),
            pl.BlockSpec((1, 4 * D), lambda i: (0, 0)),
        ],
        out_specs=[
            pl.BlockSpec((RB, D), lambda i: (i, 0)),
            pl.BlockSpec((RB, 2 * D), lambda i: (i, 0)),
            pl.BlockSpec((RB, D), lambda i: (i, 0)),
        ],
        out_shape=[
            jax.ShapeDtypeStruct((N, D), jnp.float32),
            jax.ShapeDtypeStruct((N, 2 * D), jnp.float32),
            jax.ShapeDtypeStruct((N, D), jnp.float32),
        ],
    )(x, wcat, bcat)


def _proj_mid_body(agg_ref, sprev_ref, w_ref, b_ref, q_ref, kv_ref, s_ref):
    a = agg_ref[0] + agg_ref[1]
    h = a[:, 8:8 + D] / (a[:, 0:1] + EPS) + sprev_ref[...]
    h = h * lax.logistic(h)
    o = jnp.dot(h, w_ref[...], preferred_element_type=jnp.float32)
    o = o + b_ref[...]
    q_ref[...] = o[:, :D]
    kv_ref[...] = o[:, D:3 * D]
    s_ref[...] = o[:, 3 * D:]


def _proj_mid(aggp, sprev, wcat, bcat):
    return pl.pallas_call(
        _proj_mid_body,
        grid=(GRID,),
        in_specs=[
            pl.BlockSpec((2, RB, WIDTH), lambda i: (0, i, 0)),
            pl.BlockSpec((RB, D), lambda i: (i, 0)),
            pl.BlockSpec((D, 4 * D), lambda i: (0, 0)),
            pl.BlockSpec((1, 4 * D), lambda i: (0, 0)),
        ],
        out_specs=[
            pl.BlockSpec((RB, D), lambda i: (i, 0)),
            pl.BlockSpec((RB, 2 * D), lambda i: (i, 0)),
            pl.BlockSpec((RB, D), lambda i: (i, 0)),
        ],
        out_shape=[
            jax.ShapeDtypeStruct((N, D), jnp.float32),
            jax.ShapeDtypeStruct((N, 2 * D), jnp.float32),
            jax.ShapeDtypeStruct((N, D), jnp.float32),
        ],
    )(aggp, sprev, wcat, bcat)


def _merge_final_body(agg_ref, sprev_ref, o_ref):
    a = agg_ref[0] + agg_ref[1]
    o_ref[...] = jnp.tanh(a[:, 8:8 + D] / (a[:, 0:1] + EPS) + sprev_ref[...])


def _merge_final(aggp, sprev):
    return pl.pallas_call(
        _merge_final_body,
        grid=(GRID,),
        in_specs=[
            pl.BlockSpec((2, RB, WIDTH), lambda i: (0, i, 0)),
            pl.BlockSpec((RB, D), lambda i: (i, 0)),
        ],
        out_specs=pl.BlockSpec((RB, D), lambda i: (i, 0)),
        out_shape=jax.ShapeDtypeStruct((N, D), jnp.float32),
    )(aggp, sprev)


_sc_mesh = plsc.VectorSubcoreMesh(core_axis_name="c", subcore_axis_name="s")


@functools.partial(
    pl.kernel,
    out_type=jax.ShapeDtypeStruct((2, N, WIDTH), jnp.float32),
    mesh=_sc_mesh,
    scratch_types=[
        pltpu.VMEM((CB,), jnp.int32),
        pltpu.VMEM((CB,), jnp.int32),
        pltpu.VMEM((CB,), jnp.int32),
        pltpu.VMEM((CB,), jnp.int32),
        pltpu.VMEM((CB, D), jnp.float32),
        pltpu.VMEM((CB, D), jnp.float32),
        pltpu.VMEM((CB, 2 * D), jnp.float32),
        pltpu.VMEM((CB, 2 * D), jnp.float32),
        pltpu.VMEM((CB, WIDTH), jnp.float32),
        pltpu.VMEM((CB, WIDTH), jnp.float32),
        pltpu.VMEM_SHARED((N, WIDTH), jnp.float32),
        pltpu.SemaphoreType.DMA,
        pltpu.SemaphoreType.DMA,
        pltpu.SemaphoreType.DMA,
        pltpu.SemaphoreType.DMA,
        pltpu.SemaphoreType.DMA,
        pltpu.SemaphoreType.DMA,
        pltpu.SemaphoreType.DMA,
        pltpu.SemaphoreType.DMA,
        pltpu.SemaphoreType.DMA,
        pltpu.SemaphoreType.DMA,
    ],
    compiler_params=pltpu.CompilerParams(
        use_tc_tiling_on_sc=False, needs_layout_passes=False),
)
def _edge_kernel(q_hbm, kv_hbm, src_hbm, dst_hbm, zero_hbm, out_hbm,
                 srcv0, dstv0, srcv1, dstv1, qrows0, qrows1,
                 kvrows0, kvrows1, orows0, orows1, aggsh,
                 semq0, semkv0, semq1, semkv1, semsc0, semsc1,
                 semsrc0, semdst0, semsrc1, semdst1):
    cid = lax.axis_index("c")
    sid = lax.axis_index("s")
    wid = sid * 2 + cid

    pltpu.sync_copy(zero_hbm, aggsh.at[pl.ds(sid * ROWS_PER_SUB, ROWS_PER_SUB)])
    plsc.subcore_barrier()

    ebase = wid * CPW * CB

    def _compute(qr, kvr, orr):
        @plsc.parallel_loop(0, CB, 1, unroll=4)
        def edge_body(e):
            acc = [qr[e, pl.ds(c * 16, 16)] * kvr[e, pl.ds(c * 16, 16)]
                   for c in range(4)]
            for c in range(4, 8):
                acc[c - 4] = acc[c - 4] + qr[e, pl.ds(c * 16, 16)] * kvr[e, pl.ds(c * 16, 16)]
            aa = jnp.sum((acc[0] + acc[1]) + (acc[2] + acc[3]))
            ex16 = jnp.exp(jnp.full((16,), aa * INV_SQRT_C, jnp.float32))
            orr[e, pl.ds(0, 16)] = ex16
            for c in range(8):
                orr[e, pl.ds(8 + c * 16, 16)] = kvr[e, pl.ds(D + c * 16, 16)] * ex16

    def _issue_gathers(sv, dv, qr, kvr, sq, skv):
        pltpu.async_copy(q_hbm.at[dv], qr, sq)
        pltpu.async_copy(kv_hbm.at[sv], kvr, skv)

    def _wait_gathers(sv, dv, qr, kvr, sq, skv):
        pltpu.make_async_copy(q_hbm.at[dv], qr, sq).wait()
        pltpu.make_async_copy(kv_hbm.at[sv], kvr, skv).wait()

    def _issue_idx(base, sv, dv, ss, sd):
        pltpu.async_copy(src_hbm.at[pl.ds(base, CB)], sv, ss)
        pltpu.async_copy(dst_hbm.at[pl.ds(base, CB)], dv, sd)

    def _wait_idx(sv, dv, ss, sd):
        pltpu.make_async_copy(src_hbm.at[pl.ds(0, CB)], sv, ss).wait()
        pltpu.make_async_copy(dst_hbm.at[pl.ds(0, CB)], dv, sd).wait()

    pltpu.sync_copy(src_hbm.at[pl.ds(ebase, CB)], srcv0)
    pltpu.sync_copy(dst_hbm.at[pl.ds(ebase, CB)], dstv0)
    _issue_gathers(srcv0, dstv0, qrows0, kvrows0, semq0, semkv0)
    pltpu.sync_copy(src_hbm.at[pl.ds(ebase + CB, CB)], srcv1)
    pltpu.sync_copy(dst_hbm.at[pl.ds(ebase + CB, CB)], dstv1)

    def pair_body(p, carry):
        not_last = p < NPAIRS - 1

        @pl.when(p > 0)
        def _():
            _wait_idx(srcv1, dstv1, semsrc1, semdst1)

        _issue_gathers(srcv1, dstv1, qrows1, kvrows1, semq1, semkv1)
        _wait_gathers(srcv0, dstv0, qrows0, kvrows0, semq0, semkv0)
        _compute(qrows0, kvrows0, orows0)
        pltpu.async_copy(orows0, aggsh.at[dstv0], semsc0, add=True)
        _wait_gathers(srcv1, dstv1, qrows1, kvrows1, semq1, semkv1)
        _compute(qrows1, kvrows1, orows1)
        pltpu.make_async_copy(orows0, aggsh.at[dstv0], semsc0).wait()
        pltpu.async_copy(orows1, aggsh.at[dstv1], semsc1, add=True)

        @pl.when(not_last)
        def _():
            _issue_idx(ebase + (2 * p + 2) * CB, srcv0, dstv0, semsrc0, semdst0)

        pltpu.make_async_copy(orows1, aggsh.at[dstv1], semsc1).wait()

        @pl.when(not_last)
        def _():
            _wait_idx(srcv0, dstv0, semsrc0, semdst0)
            _issue_gathers(srcv0, dstv0, qrows0, kvrows0, semq0, semkv0)
            _issue_idx(ebase + (2 * p + 3) * CB, srcv1, dstv1, semsrc1, semdst1)

        return carry

    lax.fori_loop(0, NPAIRS, pair_body, 0)

    plsc.subcore_barrier()
    r0 = sid * ROWS_PER_SUB
    pltpu.sync_copy(aggsh.at[pl.ds(r0, ROWS_PER_SUB)],
                    out_hbm.at[cid, pl.ds(r0, ROWS_PER_SUB)])


def kernel(x, edge_index,
           Wq1, bq1, Wk1, bk1, Wv1, bv1, Ws1, bs1,
           Wq2, bq2, Wk2, bk2, Wv2, bv2, Ws2, bs2,
           Wq3, bq3, Wk3, bk3, Wv3, bv3, Ws3, bs3):
    src = edge_index[0].astype(jnp.int32)
    dst = edge_index[1].astype(jnp.int32)
    zeros = jnp.zeros((ROWS_PER_SUB, WIDTH), jnp.float32)

    w1 = jnp.concatenate([Wq1, Wk1, Wv1, Ws1], axis=1)
    b1 = jnp.concatenate([bq1, bk1, bv1, bs1]).reshape(1, 4 * D)
    w2 = jnp.concatenate([Wq2, Wk2, Wv2, Ws2], axis=1)
    b2 = jnp.concatenate([bq2, bk2, bv2, bs2]).reshape(1, 4 * D)
    w3 = jnp.concatenate([Wq3, Wk3, Wv3, Ws3], axis=1)
    b3 = jnp.concatenate([bq3, bk3, bv3, bs3]).reshape(1, 4 * D)

    q, kv, s = _proj_first(x, w1, b1)
    aggp = _edge_kernel(q, kv, src, dst, zeros)
    q, kv, s = _proj_mid(aggp, s, w2, b2)
    aggp = _edge_kernel(q, kv, src, dst, zeros)
    q, kv, s = _proj_mid(aggp, s, w3, b3)
    aggp = _edge_kernel(q, kv, src, dst, zeros)
    return _merge_final(aggp, s)


# speedup vs baseline: 1.2709x; 1.2709x over previous
import functools
import math

import numpy as np

import jax
import jax.numpy as jnp
from jax import lax
from jax.experimental import pallas as pl
from jax.experimental.pallas import tpu as pltpu
from jax.experimental.pallas import tpu_sc as plsc

N = 10000
E = 320000
D = 128
WIDTH = 136
CB = 40
NCHUNKS = E // CB
NW = 32
CPW = NCHUNKS // NW
NPAIRS = CPW // 2
NSUB = 16
ROWS_PER_SUB = N // NSUB
INV_SQRT_C = 1.0 / math.sqrt(float(D))
EPS = 1e-16

RB = 512
GRID = (N + RB - 1) // RB


def _proj_first_body(x_ref, w_ref, b_ref, q_ref, kv_ref, s_ref):
    o = jnp.dot(x_ref[...], w_ref[...], preferred_element_type=jnp.float32)
    o = o + b_ref[...]
    q_ref[...] = o[:, :D].astype(jnp.bfloat16)
    kv_ref[...] = o[:, D:3 * D].astype(jnp.bfloat16)
    s_ref[...] = o[:, 3 * D:]


def _proj_first(x, wcat, bcat):
    return pl.pallas_call(
        _proj_first_body,
        grid=(GRID,),
        in_specs=[
            pl.BlockSpec((RB, D), lambda i: (i, 0)),
            pl.BlockSpec((D, 4 * D), lambda i: (0, 0)),
            pl.BlockSpec((1, 4 * D), lambda i: (0, 0)),
        ],
        out_specs=[
            pl.BlockSpec((RB, D), lambda i: (i, 0)),
            pl.BlockSpec((RB, 2 * D), lambda i: (i, 0)),
            pl.BlockSpec((RB, D), lambda i: (i, 0)),
        ],
        out_shape=[
            jax.ShapeDtypeStruct((N, D), jnp.bfloat16),
            jax.ShapeDtypeStruct((N, 2 * D), jnp.bfloat16),
            jax.ShapeDtypeStruct((N, D), jnp.float32),
        ],
    )(x, wcat, bcat)


def _proj_mid_body(agg_ref, sprev_ref, p_ref, w_ref, b_ref, q_ref, kv_ref, s_ref):
    a = agg_ref[0] + agg_ref[1]
    agg = jnp.dot(a[:, 8:8 + D], p_ref[...], preferred_element_type=jnp.float32)
    h = agg / (a[:, 0:1] + EPS) + sprev_ref[...]
    h = h * lax.logistic(h)
    o = jnp.dot(h, w_ref[...], preferred_element_type=jnp.float32)
    o = o + b_ref[...]
    q_ref[...] = o[:, :D].astype(jnp.bfloat16)
    kv_ref[...] = o[:, D:3 * D].astype(jnp.bfloat16)
    s_ref[...] = o[:, 3 * D:]


def _proj_mid(aggp, sprev, perm, wcat, bcat):
    return pl.pallas_call(
        _proj_mid_body,
        grid=(GRID,),
        in_specs=[
            pl.BlockSpec((2, RB, WIDTH), lambda i: (0, i, 0)),
            pl.BlockSpec((RB, D), lambda i: (i, 0)),
            pl.BlockSpec((D, D), lambda i: (0, 0)),
            pl.BlockSpec((D, 4 * D), lambda i: (0, 0)),
            pl.BlockSpec((1, 4 * D), lambda i: (0, 0)),
        ],
        out_specs=[
            pl.BlockSpec((RB, D), lambda i: (i, 0)),
            pl.BlockSpec((RB, 2 * D), lambda i: (i, 0)),
            pl.BlockSpec((RB, D), lambda i: (i, 0)),
        ],
        out_shape=[
            jax.ShapeDtypeStruct((N, D), jnp.bfloat16),
            jax.ShapeDtypeStruct((N, 2 * D), jnp.bfloat16),
            jax.ShapeDtypeStruct((N, D), jnp.float32),
        ],
    )(aggp, sprev, perm, wcat, bcat)


def _merge_final_body(agg_ref, sprev_ref, p_ref, o_ref):
    a = agg_ref[0] + agg_ref[1]
    agg = jnp.dot(a[:, 8:8 + D], p_ref[...], preferred_element_type=jnp.float32)
    o_ref[...] = jnp.tanh(agg / (a[:, 0:1] + EPS) + sprev_ref[...])


def _merge_final(aggp, sprev, perm):
    return pl.pallas_call(
        _merge_final_body,
        grid=(GRID,),
        in_specs=[
            pl.BlockSpec((2, RB, WIDTH), lambda i: (0, i, 0)),
            pl.BlockSpec((RB, D), lambda i: (i, 0)),
            pl.BlockSpec((D, D), lambda i: (0, 0)),
        ],
        out_specs=pl.BlockSpec((RB, D), lambda i: (i, 0)),
        out_shape=jax.ShapeDtypeStruct((N, D), jnp.float32),
    )(aggp, sprev, perm)


_sc_mesh = plsc.VectorSubcoreMesh(core_axis_name="c", subcore_axis_name="s")


@functools.partial(
    pl.kernel,
    out_type=jax.ShapeDtypeStruct((2, N, WIDTH), jnp.float32),
    mesh=_sc_mesh,
    scratch_types=[
        pltpu.VMEM((CB,), jnp.int32),
        pltpu.VMEM((CB,), jnp.int32),
        pltpu.VMEM((CB,), jnp.int32),
        pltpu.VMEM((CB,), jnp.int32),
        pltpu.VMEM((CB, D), jnp.bfloat16),
        pltpu.VMEM((CB, D), jnp.bfloat16),
        pltpu.VMEM((CB, 2 * D), jnp.bfloat16),
        pltpu.VMEM((CB, 2 * D), jnp.bfloat16),
        pltpu.VMEM((CB, WIDTH), jnp.float32),
        pltpu.VMEM((CB, WIDTH), jnp.float32),
        pltpu.VMEM_SHARED((N, WIDTH), jnp.float32),
        pltpu.SemaphoreType.DMA,
        pltpu.SemaphoreType.DMA,
        pltpu.SemaphoreType.DMA,
        pltpu.SemaphoreType.DMA,
        pltpu.SemaphoreType.DMA,
        pltpu.SemaphoreType.DMA,
        pltpu.SemaphoreType.DMA,
        pltpu.SemaphoreType.DMA,
        pltpu.SemaphoreType.DMA,
        pltpu.SemaphoreType.DMA,
    ],
    compiler_params=pltpu.CompilerParams(
        use_tc_tiling_on_sc=False, needs_layout_passes=False),
)
def _edge_kernel(q_hbm, kv_hbm, src_hbm, dst_hbm, zero_hbm, out_hbm,
                 srcv0, dstv0, srcv1, dstv1, qrows0, qrows1,
                 kvrows0, kvrows1, orows0, orows1, aggsh,
                 semq0, semkv0, semq1, semkv1, semsc0, semsc1,
                 semsrc0, semdst0, semsrc1, semdst1):
    cid = lax.axis_index("c")
    sid = lax.axis_index("s")
    wid = sid * 2 + cid

    pltpu.sync_copy(zero_hbm, aggsh.at[pl.ds(sid * ROWS_PER_SUB, ROWS_PER_SUB)])
    plsc.subcore_barrier()

    ebase = wid * CPW * CB

    def _compute(qr, kvr, orr):
        @plsc.parallel_loop(0, CB, 1, unroll=2)
        def edge_body(e):
            acc0 = jnp.zeros((16,), jnp.float32)
            acc1 = jnp.zeros((16,), jnp.float32)
            for c in range(4):
                pr = qr[e, pl.ds(c * 32, 32)] * kvr[e, pl.ds(c * 32, 32)]
                u0, u1 = plsc.unpack(pr, format=plsc.PackFormat.INTERLEAVED)
                acc0 = acc0 + u0
                acc1 = acc1 + u1
            aa = jnp.sum(acc0 + acc1)
            ex16 = jnp.exp(jnp.full((16,), aa * INV_SQRT_C, jnp.float32))
            orr[e, pl.ds(0, 16)] = ex16
            for c in range(4):
                v0, v1 = plsc.unpack(kvr[e, pl.ds(D + c * 32, 32)],
                                     format=plsc.PackFormat.INTERLEAVED)
                orr[e, pl.ds(8 + c * 32, 16)] = v0 * ex16
                orr[e, pl.ds(8 + c * 32 + 16, 16)] = v1 * ex16

    def _issue_gathers(sv, dv, qr, kvr, sq, skv):
        pltpu.async_copy(q_hbm.at[dv], qr, sq)
        pltpu.async_copy(kv_hbm.at[sv], kvr, skv)

    def _wait_gathers(sv, dv, qr, kvr, sq, skv):
        pltpu.make_async_copy(q_hbm.at[dv], qr, sq).wait()
        pltpu.make_async_copy(kv_hbm.at[sv], kvr, skv).wait()

    def _issue_idx(base, sv, dv, ss, sd):
        pltpu.async_copy(src_hbm.at[pl.ds(base, CB)], sv, ss)
        pltpu.async_copy(dst_hbm.at[pl.ds(base, CB)], dv, sd)

    def _wait_idx(sv, dv, ss, sd):
        pltpu.make_async_copy(src_hbm.at[pl.ds(0, CB)], sv, ss).wait()
        pltpu.make_async_copy(dst_hbm.at[pl.ds(0, CB)], dv, sd).wait()

    pltpu.sync_copy(src_hbm.at[pl.ds(ebase, CB)], srcv0)
    pltpu.sync_copy(dst_hbm.at[pl.ds(ebase, CB)], dstv0)
    _issue_gathers(srcv0, dstv0, qrows0, kvrows0, semq0, semkv0)
    pltpu.sync_copy(src_hbm.at[pl.ds(ebase + CB, CB)], srcv1)
    pltpu.sync_copy(dst_hbm.at[pl.ds(ebase + CB, CB)], dstv1)

    def pair_body(p, carry):
        not_last = p < NPAIRS - 1

        @pl.when(p > 0)
        def _():
            _wait_idx(srcv1, dstv1, semsrc1, semdst1)

        _issue_gathers(srcv1, dstv1, qrows1, kvrows1, semq1, semkv1)
        _wait_gathers(srcv0, dstv0, qrows0, kvrows0, semq0, semkv0)
        _compute(qrows0, kvrows0, orows0)
        pltpu.async_copy(orows0, aggsh.at[dstv0], semsc0, add=True)
        _wait_gathers(srcv1, dstv1, qrows1, kvrows1, semq1, semkv1)
        _compute(qrows1, kvrows1, orows1)
        pltpu.make_async_copy(orows0, aggsh.at[dstv0], semsc0).wait()
        pltpu.async_copy(orows1, aggsh.at[dstv1], semsc1, add=True)

        @pl.when(not_last)
        def _():
            _issue_idx(ebase + (2 * p + 2) * CB, srcv0, dstv0, semsrc0, semdst0)

        pltpu.make_async_copy(orows1, aggsh.at[dstv1], semsc1).wait()

        @pl.when(not_last)
        def _():
            _wait_idx(srcv0, dstv0, semsrc0, semdst0)
            _issue_gathers(srcv0, dstv0, qrows0, kvrows0, semq0, semkv0)
            _issue_idx(ebase + (2 * p + 3) * CB, srcv1, dstv1, semsrc1, semdst1)

        return carry

    lax.fori_loop(0, NPAIRS, pair_body, 0)

    plsc.subcore_barrier()
    r0 = sid * ROWS_PER_SUB
    pltpu.sync_copy(aggsh.at[pl.ds(r0, ROWS_PER_SUB)],
                    out_hbm.at[cid, pl.ds(r0, ROWS_PER_SUB)])


def _perm_matrix():
    p = np.zeros((D, D), np.float32)
    for c in range(4):
        for b in range(2):
            for i in range(16):
                p[32 * c + 16 * b + i, 32 * c + 2 * i + b] = 1.0
    return p


_PERM = _perm_matrix()


def kernel(x, edge_index,
           Wq1, bq1, Wk1, bk1, Wv1, bv1, Ws1, bs1,
           Wq2, bq2, Wk2, bk2, Wv2, bv2, Ws2, bs2,
           Wq3, bq3, Wk3, bk3, Wv3, bv3, Ws3, bs3):
    src = edge_index[0].astype(jnp.int32)
    dst = edge_index[1].astype(jnp.int32)
    zeros = jnp.zeros((ROWS_PER_SUB, WIDTH), jnp.float32)

    w1 = jnp.concatenate([Wq1, Wk1, Wv1, Ws1], axis=1)
    b1 = jnp.concatenate([bq1, bk1, bv1, bs1]).reshape(1, 4 * D)
    w2 = jnp.concatenate([Wq2, Wk2, Wv2, Ws2], axis=1)
    b2 = jnp.concatenate([bq2, bk2, bv2, bs2]).reshape(1, 4 * D)
    w3 = jnp.concatenate([Wq3, Wk3, Wv3, Ws3], axis=1)
    b3 = jnp.concatenate([bq3, bk3, bv3, bs3]).reshape(1, 4 * D)

    q, kv, s = _proj_first(x, w1, b1)
    aggp = _edge_kernel(q, kv, src, dst, zeros)
    q, kv, s = _proj_mid(aggp, s, _PERM, w2, b2)
    aggp = _edge_kernel(q, kv, src, dst, zeros)
    q, kv, s = _proj_mid(aggp, s, _PERM, w3, b3)
    aggp = _edge_kernel(q, kv, src, dst, zeros)
    return _merge_final(aggp, s, _PERM)


# speedup vs baseline: 1.2730x; 1.0016x over previous
import functools
import math

import numpy as np

import jax
import jax.numpy as jnp
from jax import lax
from jax.experimental import pallas as pl
from jax.experimental.pallas import tpu as pltpu
from jax.experimental.pallas import tpu_sc as plsc

N = 10000
E = 320000
D = 128
WIDTH = 136
CB = 40
NCHUNKS = E // CB
NW = 32
CPW = NCHUNKS // NW
NPAIRS = CPW // 2
NSUB = 16
ROWS_PER_SUB = N // NSUB
INV_SQRT_C = 1.0 / math.sqrt(float(D))
EPS = 1e-16

RB = 512
GRID = (N + RB - 1) // RB


def _proj_first_body(x_ref, w_ref, b_ref, q_ref, kv_ref, s_ref):
    o = jnp.dot(x_ref[...], w_ref[...], preferred_element_type=jnp.float32)
    o = o + b_ref[...]
    q_ref[...] = o[:, :D].astype(jnp.bfloat16)
    kv_ref[...] = o[:, D:3 * D].astype(jnp.bfloat16)
    s_ref[...] = o[:, 3 * D:]


def _proj_first(x, wcat, bcat):
    return pl.pallas_call(
        _proj_first_body,
        grid=(GRID,),
        in_specs=[
            pl.BlockSpec((RB, D), lambda i: (i, 0)),
            pl.BlockSpec((D, 4 * D), lambda i: (0, 0)),
            pl.BlockSpec((1, 4 * D), lambda i: (0, 0)),
        ],
        out_specs=[
            pl.BlockSpec((RB, D), lambda i: (i, 0)),
            pl.BlockSpec((RB, 2 * D), lambda i: (i, 0)),
            pl.BlockSpec((RB, D), lambda i: (i, 0)),
        ],
        out_shape=[
            jax.ShapeDtypeStruct((N, D), jnp.bfloat16),
            jax.ShapeDtypeStruct((N, 2 * D), jnp.bfloat16),
            jax.ShapeDtypeStruct((N, D), jnp.float32),
        ],
    )(x, wcat, bcat)


def _proj_mid_body(agg_ref, sprev_ref, p_ref, w_ref, b_ref, q_ref, kv_ref, s_ref):
    a = agg_ref[0] + agg_ref[1]
    agg = jnp.dot(a[:, 8:8 + D], p_ref[...], preferred_element_type=jnp.float32)
    h = agg / (a[:, 0:1] + EPS) + sprev_ref[...]
    h = h * lax.logistic(h)
    o = jnp.dot(h, w_ref[...], preferred_element_type=jnp.float32)
    o = o + b_ref[...]
    q_ref[...] = o[:, :D].astype(jnp.bfloat16)
    kv_ref[...] = o[:, D:3 * D].astype(jnp.bfloat16)
    s_ref[...] = o[:, 3 * D:]


def _proj_mid(aggp, sprev, perm, wcat, bcat):
    return pl.pallas_call(
        _proj_mid_body,
        grid=(GRID,),
        in_specs=[
            pl.BlockSpec((2, RB, WIDTH), lambda i: (0, i, 0)),
            pl.BlockSpec((RB, D), lambda i: (i, 0)),
            pl.BlockSpec((D, D), lambda i: (0, 0)),
            pl.BlockSpec((D, 4 * D), lambda i: (0, 0)),
            pl.BlockSpec((1, 4 * D), lambda i: (0, 0)),
        ],
        out_specs=[
            pl.BlockSpec((RB, D), lambda i: (i, 0)),
            pl.BlockSpec((RB, 2 * D), lambda i: (i, 0)),
            pl.BlockSpec((RB, D), lambda i: (i, 0)),
        ],
        out_shape=[
            jax.ShapeDtypeStruct((N, D), jnp.bfloat16),
            jax.ShapeDtypeStruct((N, 2 * D), jnp.bfloat16),
            jax.ShapeDtypeStruct((N, D), jnp.float32),
        ],
    )(aggp, sprev, perm, wcat, bcat)


def _merge_final_body(agg_ref, sprev_ref, p_ref, o_ref):
    a = agg_ref[0] + agg_ref[1]
    agg = jnp.dot(a[:, 8:8 + D], p_ref[...], preferred_element_type=jnp.float32)
    o_ref[...] = jnp.tanh(agg / (a[:, 0:1] + EPS) + sprev_ref[...])


def _merge_final(aggp, sprev, perm):
    return pl.pallas_call(
        _merge_final_body,
        grid=(GRID,),
        in_specs=[
            pl.BlockSpec((2, RB, WIDTH), lambda i: (0, i, 0)),
            pl.BlockSpec((RB, D), lambda i: (i, 0)),
            pl.BlockSpec((D, D), lambda i: (0, 0)),
        ],
        out_specs=pl.BlockSpec((RB, D), lambda i: (i, 0)),
        out_shape=jax.ShapeDtypeStruct((N, D), jnp.float32),
    )(aggp, sprev, perm)


_sc_mesh = plsc.VectorSubcoreMesh(core_axis_name="c", subcore_axis_name="s")


@functools.partial(
    pl.kernel,
    out_type=jax.ShapeDtypeStruct((2, N, WIDTH), jnp.float32),
    mesh=_sc_mesh,
    scratch_types=[
        pltpu.VMEM((CB,), jnp.int32),
        pltpu.VMEM((CB,), jnp.int32),
        pltpu.VMEM((CB,), jnp.int32),
        pltpu.VMEM((CB,), jnp.int32),
        pltpu.VMEM((CB, D), jnp.bfloat16),
        pltpu.VMEM((CB, D), jnp.bfloat16),
        pltpu.VMEM((CB, 2 * D), jnp.bfloat16),
        pltpu.VMEM((CB, 2 * D), jnp.bfloat16),
        pltpu.VMEM((CB, WIDTH), jnp.float32),
        pltpu.VMEM((CB, WIDTH), jnp.float32),
        pltpu.VMEM_SHARED((N, WIDTH), jnp.float32),
        pltpu.SemaphoreType.DMA,
        pltpu.SemaphoreType.DMA,
        pltpu.SemaphoreType.DMA,
        pltpu.SemaphoreType.DMA,
        pltpu.SemaphoreType.DMA,
        pltpu.SemaphoreType.DMA,
        pltpu.SemaphoreType.DMA,
        pltpu.SemaphoreType.DMA,
        pltpu.SemaphoreType.DMA,
        pltpu.SemaphoreType.DMA,
    ],
    compiler_params=pltpu.CompilerParams(
        use_tc_tiling_on_sc=False, needs_layout_passes=False),
)
def _edge_kernel(q_hbm, kv_hbm, src_hbm, dst_hbm, zero_hbm, out_hbm,
                 srcv0, dstv0, srcv1, dstv1, qrows0, qrows1,
                 kvrows0, kvrows1, orows0, orows1, aggsh,
                 semq0, semkv0, semq1, semkv1, semsc0, semsc1,
                 semsrc0, semdst0, semsrc1, semdst1):
    cid = lax.axis_index("c")
    sid = lax.axis_index("s")
    wid = sid * 2 + cid

    pltpu.sync_copy(zero_hbm, aggsh.at[pl.ds(sid * ROWS_PER_SUB, ROWS_PER_SUB)])
    plsc.subcore_barrier()

    ebase = wid * CPW * CB

    def _compute(qr, kvr, orr):
        @plsc.parallel_loop(0, CB, 1, unroll=4)
        def edge_body(e):
            acc0 = jnp.zeros((16,), jnp.float32)
            acc1 = jnp.zeros((16,), jnp.float32)
            for c in range(4):
                pr = qr[e, pl.ds(c * 32, 32)] * kvr[e, pl.ds(c * 32, 32)]
                u0, u1 = plsc.unpack(pr, format=plsc.PackFormat.INTERLEAVED)
                acc0 = acc0 + u0
                acc1 = acc1 + u1
            aa = jnp.sum(acc0 + acc1)
            ex16 = jnp.exp(jnp.full((16,), aa * INV_SQRT_C, jnp.float32))
            orr[e, pl.ds(0, 16)] = ex16
            for c in range(4):
                v0, v1 = plsc.unpack(kvr[e, pl.ds(D + c * 32, 32)],
                                     format=plsc.PackFormat.INTERLEAVED)
                orr[e, pl.ds(8 + c * 32, 16)] = v0 * ex16
                orr[e, pl.ds(8 + c * 32 + 16, 16)] = v1 * ex16

    def _issue_gathers(sv, dv, qr, kvr, sq, skv):
        pltpu.async_copy(q_hbm.at[dv], qr, sq)
        pltpu.async_copy(kv_hbm.at[sv], kvr, skv)

    def _wait_gathers(sv, dv, qr, kvr, sq, skv):
        pltpu.make_async_copy(q_hbm.at[dv], qr, sq).wait()
        pltpu.make_async_copy(kv_hbm.at[sv], kvr, skv).wait()

    def _issue_idx(base, sv, dv, ss, sd):
        pltpu.async_copy(src_hbm.at[pl.ds(base, CB)], sv, ss)
        pltpu.async_copy(dst_hbm.at[pl.ds(base, CB)], dv, sd)

    def _wait_idx(sv, dv, ss, sd):
        pltpu.make_async_copy(src_hbm.at[pl.ds(0, CB)], sv, ss).wait()
        pltpu.make_async_copy(dst_hbm.at[pl.ds(0, CB)], dv, sd).wait()

    pltpu.sync_copy(src_hbm.at[pl.ds(ebase, CB)], srcv0)
    pltpu.sync_copy(dst_hbm.at[pl.ds(ebase, CB)], dstv0)
    _issue_gathers(srcv0, dstv0, qrows0, kvrows0, semq0, semkv0)
    pltpu.sync_copy(src_hbm.at[pl.ds(ebase + CB, CB)], srcv1)
    pltpu.sync_copy(dst_hbm.at[pl.ds(ebase + CB, CB)], dstv1)

    def pair_body(p, carry):
        not_last = p < NPAIRS - 1

        @pl.when(p > 0)
        def _():
            _wait_idx(srcv1, dstv1, semsrc1, semdst1)

        _issue_gathers(srcv1, dstv1, qrows1, kvrows1, semq1, semkv1)
        _wait_gathers(srcv0, dstv0, qrows0, kvrows0, semq0, semkv0)
        _compute(qrows0, kvrows0, orows0)
        pltpu.async_copy(orows0, aggsh.at[dstv0], semsc0, add=True)
        _wait_gathers(srcv1, dstv1, qrows1, kvrows1, semq1, semkv1)
        _compute(qrows1, kvrows1, orows1)
        pltpu.make_async_copy(orows0, aggsh.at[dstv0], semsc0).wait()
        pltpu.async_copy(orows1, aggsh.at[dstv1], semsc1, add=True)

        @pl.when(not_last)
        def _():
            _issue_idx(ebase + (2 * p + 2) * CB, srcv0, dstv0, semsrc0, semdst0)

        pltpu.make_async_copy(orows1, aggsh.at[dstv1], semsc1).wait()

        @pl.when(not_last)
        def _():
            _wait_idx(srcv0, dstv0, semsrc0, semdst0)
            _issue_gathers(srcv0, dstv0, qrows0, kvrows0, semq0, semkv0)
            _issue_idx(ebase + (2 * p + 3) * CB, srcv1, dstv1, semsrc1, semdst1)

        return carry

    lax.fori_loop(0, NPAIRS, pair_body, 0)

    plsc.subcore_barrier()
    r0 = sid * ROWS_PER_SUB
    pltpu.sync_copy(aggsh.at[pl.ds(r0, ROWS_PER_SUB)],
                    out_hbm.at[cid, pl.ds(r0, ROWS_PER_SUB)])


def _perm_matrix():
    p = np.zeros((D, D), np.float32)
    for c in range(4):
        for b in range(2):
            for i in range(16):
                p[32 * c + 16 * b + i, 32 * c + 2 * i + b] = 1.0
    return p


_PERM = _perm_matrix()


def kernel(x, edge_index,
           Wq1, bq1, Wk1, bk1, Wv1, bv1, Ws1, bs1,
           Wq2, bq2, Wk2, bk2, Wv2, bv2, Ws2, bs2,
           Wq3, bq3, Wk3, bk3, Wv3, bv3, Ws3, bs3):
    src = edge_index[0].astype(jnp.int32)
    dst = edge_index[1].astype(jnp.int32)
    zeros = jnp.zeros((ROWS_PER_SUB, WIDTH), jnp.float32)

    w1 = jnp.concatenate([Wq1, Wk1, Wv1, Ws1], axis=1)
    b1 = jnp.concatenate([bq1, bk1, bv1, bs1]).reshape(1, 4 * D)
    w2 = jnp.concatenate([Wq2, Wk2, Wv2, Ws2], axis=1)
    b2 = jnp.concatenate([bq2, bk2, bv2, bs2]).reshape(1, 4 * D)
    w3 = jnp.concatenate([Wq3, Wk3, Wv3, Ws3], axis=1)
    b3 = jnp.concatenate([bq3, bk3, bv3, bs3]).reshape(1, 4 * D)

    q, kv, s = _proj_first(x, w1, b1)
    aggp = _edge_kernel(q, kv, src, dst, zeros)
    q, kv, s = _proj_mid(aggp, s, _PERM, w2, b2)
    aggp = _edge_kernel(q, kv, src, dst, zeros)
    q, kv, s = _proj_mid(aggp, s, _PERM, w3, b3)
    aggp = _edge_kernel(q, kv, src, dst, zeros)
    return _merge_final(aggp, s, _PERM)
